# Initial kernel scaffold; baseline (speedup 1.0000x reference)
import jax
import jax.numpy as jnp
from jax.experimental import pallas as pl


def kernel(words, table):
    raise NotImplementedError("write your pallas kernel here")


# speedup vs baseline: 1.8804x; 1.8804x over previous
import functools

import jax
import jax.numpy as jnp
from jax import lax
from jax.experimental import pallas as pl
from jax.experimental.pallas import tpu as pltpu
from jax.experimental.pallas import tpu_sc as plsc

D = 64
NC, NS = 2, 16
NW = NC * NS
GROUP = 128
K = 4
CHUNK = K * GROUP
TOTAL = 16384 * 50
PER_W = TOTAL // NW
N_GROUPS = PER_W // GROUP
N_CHUNKS = PER_W // CHUNK
N_PAIRS = N_CHUNKS // 2

_mesh = plsc.VectorSubcoreMesh(core_axis_name="c", subcore_axis_name="s")


@functools.partial(
    pl.kernel,
    out_type=jax.ShapeDtypeStruct((TOTAL, D), jnp.float32),
    mesh=_mesh,
    scratch_types=[
        pltpu.VMEM((N_GROUPS, GROUP), jnp.int32),
        pltpu.VMEM((CHUNK, D), jnp.float32),
        pltpu.VMEM((CHUNK, D), jnp.float32),
        pltpu.SemaphoreType.DMA,
        pltpu.SemaphoreType.DMA,
        pltpu.SemaphoreType.DMA,
    ],
    compiler_params=pltpu.CompilerParams(use_tc_tiling_on_sc=False),
)
def _sc_gather(words_hbm, table_hbm, out_hbm, idx_v, rows0, rows1,
               gsem, osem0, osem1):
    wid = lax.axis_index("s") * NC + lax.axis_index("c")
    base = wid * PER_W
    pltpu.sync_copy(words_hbm.at[pl.ds(wid * N_GROUPS, N_GROUPS)], idx_v)

    rows = (rows0, rows1)
    osems = (osem0, osem1)

    def pair(i, _):
        for b in range(2):
            g = i * 2 + b

            @pl.when(i > 0)
            def _wait_prev():
                pltpu.make_async_copy(
                    rows[b], out_hbm.at[pl.ds(base, CHUNK)], osems[b]
                ).wait()

            fired = []
            for j in range(K):
                fired.append(pltpu.async_copy(
                    table_hbm.at[idx_v.at[g * K + j]],
                    rows[b].at[pl.ds(j * GROUP, GROUP)],
                    gsem,
                ))
            for c in fired:
                c.wait()

            pltpu.async_copy(
                rows[b],
                out_hbm.at[pl.ds(base + g * CHUNK, CHUNK)],
                osems[b],
            )
        return ()

    lax.fori_loop(0, N_PAIRS, pair, (), unroll=False)

    for b in range(2):
        pltpu.make_async_copy(
            rows[b], out_hbm.at[pl.ds(base, CHUNK)], osems[b]
        ).wait()


def kernel(words, table):
    flat = words.reshape(TOTAL // GROUP, GROUP)
    out = _sc_gather(flat, table)
    return out.reshape(words.shape[0], words.shape[1], D)
